# baseline (speedup 1.0000x reference)
import functools

import jax
import jax.numpy as jnp
from jax import lax
from jax.experimental import pallas as pl
from jax.experimental.pallas import tpu as pltpu
from jax.experimental.pallas import tpu_sc as plsc

N = 10000
E = 160000
F = 128
G = 4
V = 8

NPAD = 10240
NW = 32
PER_W = 4992
CHUNK_ROWS = (5, 5, 5, 5, 5, 5, 5, 4)
CH = 640
NCHUNK = len(CHUNK_ROWS)
TAIL_BASE = NW * PER_W
TAIL_CH = 128
ROWS_PER_SUB = NPAD // 16

_OFFDIAG = [(0, 1), (0, 2), (0, 3), (1, 2), (1, 3), (2, 3)]


def _iota16():
    return lax.iota(jnp.int32, 16)


def _c16(c):
    return jnp.full((16,), c, jnp.int32)


def _round_bf16(x):
    u = plsc.bitcast(x, jnp.uint32)
    lsb = (u >> jnp.uint32(16)) & jnp.uint32(1)
    r = (u + jnp.uint32(0x7FFF) + lsb) & jnp.uint32(0xFFFF0000)
    return plsc.bitcast(r, jnp.float32)


def _sc_edge_body(idx_hbm, gs_hbm, gv_hbm, out_hbm,
                  gs_v0, gs_v1, gv_v0, gv_v1, idx_v0, idx_v1,
                  rows_v0, rows_v1, zbuf, acc,
                  sem_in0, sem_in1, sem_sc0, sem_sc1):
    cid = lax.axis_index("c")
    sid = lax.axis_index("s")
    wid = cid * 16 + sid
    gs_b = (gs_v0, gs_v1)
    gv_b = (gv_v0, gv_v1)
    idx_b = (idx_v0, idx_v1)
    rows_b = (rows_v0, rows_v1)
    sem_in = (sem_in0, sem_in1)
    sem_sc = (sem_sc0, sem_sc1)
    zero16 = jnp.zeros((16,), jnp.float32)

    def zbody(i, carry):
        plsc.store_scatter(zbuf, [lax.broadcast(i, (16,)), _iota16()], zero16)
        return carry

    lax.fori_loop(0, ROWS_PER_SUB, zbody, 0)
    pltpu.sync_copy(zbuf, acc.at[pl.ds(sid * ROWS_PER_SUB, ROWS_PER_SUB)])
    plsc.subcore_barrier()

    def start_in(base, nrow, b):
        ds = [
            pltpu.async_copy(gs_hbm.at[pl.ds(base, nrow * 128)],
                             gs_b[b].at[pl.ds(0, nrow * 128)], sem_in[b]),
            pltpu.async_copy(gv_hbm.at[pl.ds(base, nrow * 128)],
                             gv_b[b].at[pl.ds(0, nrow * 128)], sem_in[b]),
        ]
        for j in range(nrow):
            ds.append(pltpu.async_copy(
                idx_hbm.at[1, pl.ds(base + j * 128, 128)],
                idx_b[b].at[j], sem_in[b]))
        return ds

    def compute(nvec, b, rows_r):
        gs_r, gv_r = gs_b[b], gv_b[b]

        def vbody(v, carry):
            rows = v * 16 + _iota16()
            g = [_round_bf16(plsc.load_gather(gv_r, [rows, _c16(d), _c16(j)]))
                 for d in range(3) for j in range(4)]
            s = (plsc.load_gather(gs_r, [rows, _c16(0)])
                 + plsc.load_gather(gs_r, [rows, _c16(1)])
                 + plsc.load_gather(gs_r, [rows, _c16(2)])
                 + plsc.load_gather(gs_r, [rows, _c16(3)]))
            plsc.store_scatter(rows_r, [rows, _c16(0)], s)
            for j in range(4):
                m = g[j] * g[j] + g[4 + j] * g[4 + j] + g[8 + j] * g[8 + j]
                plsc.store_scatter(rows_r, [rows, _c16(1 + j)], m)
            for col, (j, k) in enumerate(_OFFDIAG):
                m = g[j] * g[k] + g[4 + j] * g[4 + k] + g[8 + j] * g[8 + k]
                plsc.store_scatter(rows_r, [rows, _c16(5 + col)], m)
            return carry

        lax.fori_loop(0, nvec, vbody, 0)

    chunk_base = [wid * PER_W + sum(CHUNK_ROWS[:c]) * 128
                  for c in range(NCHUNK)]
    pend_in = {0: start_in(chunk_base[0], CHUNK_ROWS[0], 0)}
    pend_sc = {}
    for c in range(NCHUNK):
        b = c % 2
        nrow = CHUNK_ROWS[c]
        if c + 1 < NCHUNK:
            pend_in[c + 1] = start_in(chunk_base[c + 1], CHUNK_ROWS[c + 1],
                                      (c + 1) % 2)
        for d in pend_in.pop(c):
            d.wait()
        if c >= 2:
            for d in pend_sc.pop(c - 2):
                d.wait()
        compute(nrow * 8, b, rows_b[b])
        pend_sc[c] = [
            pltpu.async_copy(rows_b[b].at[pl.ds(j * 128, 128)],
                             acc.at[idx_b[b].at[j]], sem_sc[b], add=True)
            for j in range(nrow)
        ]

    for d in pend_sc.pop(NCHUNK - 2):
        d.wait()

    @pl.when(wid < 2)
    def _tail():
        tb = TAIL_BASE + wid * TAIL_CH
        pltpu.sync_copy(gs_hbm.at[pl.ds(tb, TAIL_CH)],
                        gs_b[0].at[pl.ds(0, TAIL_CH)])
        pltpu.sync_copy(gv_hbm.at[pl.ds(tb, TAIL_CH)],
                        gv_b[0].at[pl.ds(0, TAIL_CH)])
        pltpu.sync_copy(idx_hbm.at[1, pl.ds(tb, TAIL_CH)], idx_b[0].at[0])
        compute(TAIL_CH // 16, 0, rows_b[0])
        pltpu.sync_copy(rows_b[0].at[pl.ds(0, TAIL_CH)],
                        acc.at[idx_b[0].at[0]], add=True)

    for d in pend_sc.pop(NCHUNK - 1):
        d.wait()
    plsc.subcore_barrier()
    pltpu.sync_copy(acc.at[pl.ds(sid * ROWS_PER_SUB, ROWS_PER_SUB)],
                    out_hbm.at[cid, pl.ds(sid * ROWS_PER_SUB, ROWS_PER_SUB)])


@functools.lru_cache(maxsize=1)
def _sc_edge():
    return pl.kernel(
        _sc_edge_body,
        out_type=jax.ShapeDtypeStruct((2, NPAD, 16), jnp.float32),
        mesh=plsc.VectorSubcoreMesh(core_axis_name="c", subcore_axis_name="s",
                                    num_cores=2, num_subcores=16),
        compiler_params=pltpu.CompilerParams(needs_layout_passes=False,
                                             use_tc_tiling_on_sc=False),
        scratch_types=[
            pltpu.VMEM((CH, 4), jnp.float32),
            pltpu.VMEM((CH, 4), jnp.float32),
            pltpu.VMEM((CH, 3, 4), jnp.float32),
            pltpu.VMEM((CH, 3, 4), jnp.float32),
            pltpu.VMEM((CH // 128, 128), jnp.int32),
            pltpu.VMEM((CH // 128, 128), jnp.int32),
            pltpu.VMEM((CH, 16), jnp.float32),
            pltpu.VMEM((CH, 16), jnp.float32),
            pltpu.VMEM((ROWS_PER_SUB, 16), jnp.float32),
            pltpu.VMEM_SHARED((NPAD, 16), jnp.float32),
            pltpu.SemaphoreType.DMA,
            pltpu.SemaphoreType.DMA,
            pltpu.SemaphoreType.DMA,
            pltpu.SemaphoreType.DMA,
        ],
    )


def _tc_node_body(a_ref, q_ref, acc0_ref, acc1_ref, agh_ref, w1_ref, b1_ref,
                  w2_ref, b2_ref, w3_ref, b3_ref, out_ref):
    a = a_ref[:, :]
    ac = acc0_ref[:, :] + acc1_ref[:, :]
    s = ac[:, 0:1]
    radial = a * s
    rq = q_ref[:, :] * s

    bf = jnp.bfloat16
    t = jnp.dot(a.astype(bf), agh_ref[:, :].astype(bf),
                preferred_element_type=jnp.float32)
    t = t.astype(bf).astype(jnp.float32)
    tg = [t[:, j * 8:(j + 1) * 8] for j in range(4)]
    ve = ac[:, 1:2] * tg[0] * tg[0]
    for j in range(1, 4):
        ve = ve + ac[:, 1 + j:2 + j] * tg[j] * tg[j]
    for col, (j, k) in enumerate(_OFFDIAG):
        ve = ve + 2.0 * ac[:, 5 + col:6 + col] * tg[j] * tg[k]

    w1 = w1_ref[:, :]
    pre1 = (jnp.dot(radial.astype(bf), w1[0:128, :].astype(bf),
                    preferred_element_type=jnp.float32)
            + jnp.dot(ve.astype(bf), w1[128:136, :].astype(bf),
                      preferred_element_type=jnp.float32)
            + rq.astype(bf).astype(jnp.float32) * w1[136:137, :]
            + b1_ref[:, :])
    h1 = jax.nn.gelu(pre1)
    h2 = jax.nn.gelu(jnp.dot(h1.astype(bf), w2_ref[:, :].astype(bf),
                             preferred_element_type=jnp.float32) + b2_ref[:, :])
    out_ref[:, :] = jnp.dot(h2.astype(bf), w3_ref[:, :].astype(bf),
                            preferred_element_type=jnp.float32) + b3_ref[:, :]


def _tc_node(a, q, acc0, acc1, agh2, w1, b1, w2, b2, w3, b3):
    R = 512
    grid = (NPAD // R,)
    return pl.pallas_call(
        _tc_node_body,
        grid=grid,
        in_specs=[
            pl.BlockSpec((R, F), lambda i: (i, 0)),
            pl.BlockSpec((R, 1), lambda i: (i, 0)),
            pl.BlockSpec((R, 16), lambda i: (i, 0)),
            pl.BlockSpec((R, 16), lambda i: (i, 0)),
            pl.BlockSpec((F, G * V), lambda i: (0, 0)),
            pl.BlockSpec((F + 2 * V + 1, F), lambda i: (0, 0)),
            pl.BlockSpec((1, F), lambda i: (0, 0)),
            pl.BlockSpec((F, F), lambda i: (0, 0)),
            pl.BlockSpec((1, F), lambda i: (0, 0)),
            pl.BlockSpec((F, F + 2), lambda i: (0, 0)),
            pl.BlockSpec((1, F + 2), lambda i: (0, 0)),
        ],
        out_specs=pl.BlockSpec((R, F + 2), lambda i: (i, 0)),
        out_shape=jax.ShapeDtypeStruct((NPAD, F + 2), jnp.float32),
    )(a, q, acc0, acc1, agh2, w1, b1, w2, b2, w3, b3)


def kernel(atomic_embedding, partial_charges, pair_indices, gs, gv, agh,
           W1, b1, W2, b2, W3, b3):
    acc = _sc_edge()(pair_indices, gs, gv)

    pad_n = NPAD - N
    a_p = jnp.concatenate([atomic_embedding, jnp.zeros((pad_n, F), jnp.float32)])
    q_p = jnp.concatenate([partial_charges, jnp.zeros((pad_n, 1), jnp.float32)])

    out = _tc_node(a_p, q_p, acc[0], acc[1], agh.reshape(F, G * V),
                   W1, b1.reshape(1, F), W2, b2.reshape(1, F),
                   W3, b3.reshape(1, F + 2))
    return (out[:N, 2:F + 2], out[:N, 0:1], out[:N, 1:2])


# speedup vs baseline: 1.3677x; 1.3677x over previous
import functools

import jax
import jax.numpy as jnp
from jax import lax
from jax.experimental import pallas as pl
from jax.experimental.pallas import tpu as pltpu
from jax.experimental.pallas import tpu_sc as plsc

N = 10000
E = 160000
F = 128
G = 4
V = 8

NPAD = 10240
NW = 32
PER_W = 5120
EPAD = NW * PER_W
EROWS = EPAD // 128
CROWS = 8
NCHUNK = PER_W // (CROWS * 128)
NVEC = CROWS * 8
ROWS_PER_SUB = NPAD // 16

_OFFDIAG = [(0, 1), (0, 2), (0, 3), (1, 2), (1, 3), (2, 3)]


def _iota16():
    return lax.iota(jnp.int32, 16)


def _c16(c):
    return jnp.full((16,), c, jnp.int32)


def _round_bf16(x):
    u = plsc.bitcast(x, jnp.uint32)
    lsb = (u >> jnp.uint32(16)) & jnp.uint32(1)
    r = (u + jnp.uint32(0x7FFF) + lsb) & jnp.uint32(0xFFFF0000)
    return plsc.bitcast(r, jnp.float32)


def _sc_edge_body(idx_hbm, gs_hbm, gv_hbm, out_hbm,
                  gs_v0, gs_v1, gv_v0, gv_v1, idx_v0, idx_v1,
                  rows_v0, rows_v1, zbuf, acc,
                  sem_in0, sem_in1, sem_sc0, sem_sc1):
    cid = lax.axis_index("c")
    sid = lax.axis_index("s")
    wid = cid * 16 + sid
    gs_b = (gs_v0, gs_v1)
    gv_b = (gv_v0, gv_v1)
    idx_b = (idx_v0, idx_v1)
    rows_b = (rows_v0, rows_v1)
    sem_in = (sem_in0, sem_in1)
    sem_sc = (sem_sc0, sem_sc1)
    zero16 = jnp.zeros((16,), jnp.float32)

    def zbody(i, carry):
        plsc.store_scatter(zbuf, [lax.broadcast(i, (16,)), _iota16()], zero16)
        return carry

    lax.fori_loop(0, ROWS_PER_SUB, zbody, 0)
    pltpu.sync_copy(zbuf, acc.at[pl.ds(sid * ROWS_PER_SUB, ROWS_PER_SUB)])
    plsc.subcore_barrier()

    def start_in(c, b):
        row0 = wid * (PER_W // 128) + c * CROWS
        return [
            pltpu.async_copy(gs_hbm.at[pl.ds(row0, CROWS)], gs_b[b],
                             sem_in[b]),
            pltpu.async_copy(gv_hbm.at[pl.ds(row0, CROWS)], gv_b[b],
                             sem_in[b]),
            pltpu.async_copy(idx_hbm.at[pl.ds(row0, CROWS)], idx_b[b],
                             sem_in[b]),
        ]

    def compute(b):
        gs_r, gv_r, rows_r = gs_b[b], gv_b[b], rows_b[b]

        def vbody(v, carry):
            row = lax.broadcast(v >> 3, (16,))
            sub = v & 7
            cgs = _iota16() * 4 + sub * 64
            cgv = _iota16() * 12 + sub * 192
            g = [_round_bf16(plsc.load_gather(gv_r, [row, cgv + _c16(k)]))
                 for k in range(12)]
            s = (plsc.load_gather(gs_r, [row, cgs])
                 + plsc.load_gather(gs_r, [row, cgs + _c16(1)])
                 + plsc.load_gather(gs_r, [row, cgs + _c16(2)])
                 + plsc.load_gather(gs_r, [row, cgs + _c16(3)]))
            rows = v * 16 + _iota16()
            plsc.store_scatter(rows_r, [rows, _c16(0)], s)
            for j in range(4):
                m = g[j] * g[j] + g[4 + j] * g[4 + j] + g[8 + j] * g[8 + j]
                plsc.store_scatter(rows_r, [rows, _c16(1 + j)], m)
            for col, (j, k) in enumerate(_OFFDIAG):
                m = g[j] * g[k] + g[4 + j] * g[4 + k] + g[8 + j] * g[8 + k]
                plsc.store_scatter(rows_r, [rows, _c16(5 + col)], m)
            return carry

        lax.fori_loop(0, NVEC, vbody, 0)

    pend_in = {0: start_in(0, 0)}
    pend_sc = {}
    for c in range(NCHUNK):
        b = c % 2
        if c + 1 < NCHUNK:
            pend_in[c + 1] = start_in(c + 1, (c + 1) % 2)
        for d in pend_in.pop(c):
            d.wait()
        if c >= 2:
            for d in pend_sc.pop(c - 2):
                d.wait()
        compute(b)
        pend_sc[c] = [
            pltpu.async_copy(rows_b[b].at[pl.ds(j * 128, 128)],
                             acc.at[idx_b[b].at[j]], sem_sc[b], add=True)
            for j in range(CROWS)
        ]

    for c in sorted(pend_sc):
        for d in pend_sc[c]:
            d.wait()
    plsc.subcore_barrier()
    pltpu.sync_copy(acc.at[pl.ds(sid * ROWS_PER_SUB, ROWS_PER_SUB)],
                    out_hbm.at[cid, pl.ds(sid * ROWS_PER_SUB, ROWS_PER_SUB)])


@functools.lru_cache(maxsize=1)
def _sc_edge():
    return pl.kernel(
        _sc_edge_body,
        out_type=jax.ShapeDtypeStruct((2, NPAD, 16), jnp.float32),
        mesh=plsc.VectorSubcoreMesh(core_axis_name="c", subcore_axis_name="s",
                                    num_cores=2, num_subcores=16),
        compiler_params=pltpu.CompilerParams(needs_layout_passes=False,
                                             use_tc_tiling_on_sc=False),
        scratch_types=[
            pltpu.VMEM((CROWS, 512), jnp.float32),
            pltpu.VMEM((CROWS, 512), jnp.float32),
            pltpu.VMEM((CROWS, 1536), jnp.float32),
            pltpu.VMEM((CROWS, 1536), jnp.float32),
            pltpu.VMEM((CROWS, 128), jnp.int32),
            pltpu.VMEM((CROWS, 128), jnp.int32),
            pltpu.VMEM((CROWS * 128, 16), jnp.float32),
            pltpu.VMEM((CROWS * 128, 16), jnp.float32),
            pltpu.VMEM((ROWS_PER_SUB, 16), jnp.float32),
            pltpu.VMEM_SHARED((NPAD, 16), jnp.float32),
            pltpu.SemaphoreType.DMA,
            pltpu.SemaphoreType.DMA,
            pltpu.SemaphoreType.DMA,
            pltpu.SemaphoreType.DMA,
        ],
    )


def _tc_node_body(a_ref, q_ref, acc0_ref, acc1_ref, agh_ref, w1_ref, b1_ref,
                  w2_ref, b2_ref, w3_ref, b3_ref, out_ref):
    a = a_ref[:, :]
    ac = acc0_ref[:, :] + acc1_ref[:, :]
    s = ac[:, 0:1]
    radial = a * s
    rq = q_ref[:, :] * s

    bf = jnp.bfloat16
    t = jnp.dot(a.astype(bf), agh_ref[:, :].astype(bf),
                preferred_element_type=jnp.float32)
    t = t.astype(bf).astype(jnp.float32)
    tg = [t[:, j * 8:(j + 1) * 8] for j in range(4)]
    ve = ac[:, 1:2] * tg[0] * tg[0]
    for j in range(1, 4):
        ve = ve + ac[:, 1 + j:2 + j] * tg[j] * tg[j]
    for col, (j, k) in enumerate(_OFFDIAG):
        ve = ve + 2.0 * ac[:, 5 + col:6 + col] * tg[j] * tg[k]

    w1 = w1_ref[:, :]
    pre1 = (jnp.dot(radial.astype(bf), w1[0:128, :].astype(bf),
                    preferred_element_type=jnp.float32)
            + jnp.dot(ve.astype(bf), w1[128:136, :].astype(bf),
                      preferred_element_type=jnp.float32)
            + rq.astype(bf).astype(jnp.float32) * w1[136:137, :]
            + b1_ref[:, :])
    h1 = jax.nn.gelu(pre1)
    h2 = jax.nn.gelu(jnp.dot(h1.astype(bf), w2_ref[:, :].astype(bf),
                             preferred_element_type=jnp.float32) + b2_ref[:, :])
    out_ref[:, :] = jnp.dot(h2.astype(bf), w3_ref[:, :].astype(bf),
                            preferred_element_type=jnp.float32) + b3_ref[:, :]


def _tc_node(a, q, acc0, acc1, agh2, w1, b1, w2, b2, w3, b3):
    R = 512
    grid = (NPAD // R,)
    return pl.pallas_call(
        _tc_node_body,
        grid=grid,
        in_specs=[
            pl.BlockSpec((R, F), lambda i: (i, 0)),
            pl.BlockSpec((R, 1), lambda i: (i, 0)),
            pl.BlockSpec((R, 16), lambda i: (i, 0)),
            pl.BlockSpec((R, 16), lambda i: (i, 0)),
            pl.BlockSpec((F, G * V), lambda i: (0, 0)),
            pl.BlockSpec((F + 2 * V + 1, F), lambda i: (0, 0)),
            pl.BlockSpec((1, F), lambda i: (0, 0)),
            pl.BlockSpec((F, F), lambda i: (0, 0)),
            pl.BlockSpec((1, F), lambda i: (0, 0)),
            pl.BlockSpec((F, F + 2), lambda i: (0, 0)),
            pl.BlockSpec((1, F + 2), lambda i: (0, 0)),
        ],
        out_specs=pl.BlockSpec((R, F + 2), lambda i: (i, 0)),
        out_shape=jax.ShapeDtypeStruct((NPAD, F + 2), jnp.float32),
    )(a, q, acc0, acc1, agh2, w1, b1, w2, b2, w3, b3)


def kernel(atomic_embedding, partial_charges, pair_indices, gs, gv, agh,
           W1, b1, W2, b2, W3, b3):
    pad_r = EROWS - E // 128
    gs_w = jnp.pad(gs.reshape(E // 128, 512), ((0, pad_r), (0, 0)))
    gv_w = jnp.pad(gv.reshape(E // 128, 1536), ((0, pad_r), (0, 0)))
    idx_w = jnp.pad(pair_indices[1].reshape(E // 128, 128),
                    ((0, pad_r), (0, 0)))

    acc = _sc_edge()(idx_w, gs_w, gv_w)

    pad_n = NPAD - N
    a_p = jnp.concatenate([atomic_embedding, jnp.zeros((pad_n, F), jnp.float32)])
    q_p = jnp.concatenate([partial_charges, jnp.zeros((pad_n, 1), jnp.float32)])

    out = _tc_node(a_p, q_p, acc[0], acc[1], agh.reshape(F, G * V),
                   W1, b1.reshape(1, F), W2, b2.reshape(1, F),
                   W3, b3.reshape(1, F + 2))
    return (out[:N, 2:F + 2], out[:N, 0:1], out[:N, 1:2])


# speedup vs baseline: 2.9023x; 2.1219x over previous
import functools

import jax
import jax.numpy as jnp
from jax import lax
from jax.experimental import pallas as pl
from jax.experimental.pallas import tpu as pltpu
from jax.experimental.pallas import tpu_sc as plsc

N = 10000
E = 160000
F = 128
G = 4
V = 8

NPAD = 10240
NW = 32
PER_W = 4992
CHUNK_ROWS = (5, 5, 5, 5, 5, 5, 5, 4)
CH = 640
NCHUNK = len(CHUNK_ROWS)
TAIL_BASE = NW * PER_W
TAIL_CH = 128
ROWS_PER_SUB = NPAD // 16

_OFFDIAG = [(0, 1), (0, 2), (0, 3), (1, 2), (1, 3), (2, 3)]


def _iota16():
    return lax.iota(jnp.int32, 16)


def _c16(c):
    return jnp.full((16,), c, jnp.int32)


def _round_bf16(x):
    u = plsc.bitcast(x, jnp.uint32)
    lsb = (u >> jnp.uint32(16)) & jnp.uint32(1)
    r = (u + jnp.uint32(0x7FFF) + lsb) & jnp.uint32(0xFFFF0000)
    return plsc.bitcast(r, jnp.float32)


def _sc_edge_body(idx_hbm, gs_hbm, gv_hbm, out_hbm,
                  gs_v0, gs_v1, gv_v0, gv_v1, idx_v0, idx_v1,
                  rows_v0, rows_v1, zbuf, acc,
                  sem_in0, sem_in1, sem_sc0, sem_sc1):
    cid = lax.axis_index("c")
    sid = lax.axis_index("s")
    wid = cid * 16 + sid
    gs_b = (gs_v0, gs_v1)
    gv_b = (gv_v0, gv_v1)
    idx_b = (idx_v0, idx_v1)
    rows_b = (rows_v0, rows_v1)
    sem_in = (sem_in0, sem_in1)
    sem_sc = (sem_sc0, sem_sc1)
    zero16 = jnp.zeros((16,), jnp.float32)

    def zbody(i, carry):
        plsc.store_scatter(zbuf, [lax.broadcast(i, (16,)), _iota16()], zero16)
        return carry

    lax.fori_loop(0, ROWS_PER_SUB, zbody, 0)
    pltpu.sync_copy(zbuf, acc.at[pl.ds(sid * ROWS_PER_SUB, ROWS_PER_SUB)])
    plsc.subcore_barrier()

    def start_in(ebase, nrow, b):
        return [
            pltpu.async_copy(gs_hbm.at[pl.ds(ebase, nrow * 128)],
                             gs_b[b].at[pl.ds(0, nrow * 128)], sem_in[b]),
            pltpu.async_copy(gv_hbm.at[pl.ds(ebase, nrow * 128)],
                             gv_b[b].at[pl.ds(0, nrow * 128)], sem_in[b]),
            pltpu.async_copy(idx_hbm.at[pl.ds(ebase // 128, nrow)],
                             idx_b[b].at[pl.ds(0, nrow)], sem_in[b]),
        ]

    def compute(nvec, b):
        gs_r, gv_r, rows_r = gs_b[b], gv_b[b], rows_b[b]

        def vbody(v, carry):
            rows = v * 16 + _iota16()
            g = [_round_bf16(plsc.load_gather(gv_r, [rows, _c16(k)]))
                 for k in range(12)]
            s = (plsc.load_gather(gs_r, [rows, _c16(0)])
                 + plsc.load_gather(gs_r, [rows, _c16(1)])
                 + plsc.load_gather(gs_r, [rows, _c16(2)])
                 + plsc.load_gather(gs_r, [rows, _c16(3)]))
            plsc.store_scatter(rows_r, [rows, _c16(0)], s)
            for j in range(4):
                m = g[j] * g[j] + g[4 + j] * g[4 + j] + g[8 + j] * g[8 + j]
                plsc.store_scatter(rows_r, [rows, _c16(1 + j)], m)
            for col, (j, k) in enumerate(_OFFDIAG):
                m = g[j] * g[k] + g[4 + j] * g[4 + k] + g[8 + j] * g[8 + k]
                plsc.store_scatter(rows_r, [rows, _c16(5 + col)], m)
            return carry

        lax.fori_loop(0, nvec, vbody, 0)

    chunk_base = [wid * PER_W + sum(CHUNK_ROWS[:c]) * 128
                  for c in range(NCHUNK)]
    pend_in = {0: start_in(chunk_base[0], CHUNK_ROWS[0], 0)}
    pend_sc = {}
    for c in range(NCHUNK):
        b = c % 2
        nrow = CHUNK_ROWS[c]
        if c + 1 < NCHUNK:
            pend_in[c + 1] = start_in(chunk_base[c + 1], CHUNK_ROWS[c + 1],
                                      (c + 1) % 2)
        for d in pend_in.pop(c):
            d.wait()
        if c >= 2:
            for d in pend_sc.pop(c - 2):
                d.wait()
        compute(nrow * 8, b)
        pend_sc[c] = [
            pltpu.async_copy(rows_b[b].at[pl.ds(j * 128, 128)],
                             acc.at[idx_b[b].at[j]], sem_sc[b], add=True)
            for j in range(nrow)
        ]

    for d in pend_sc.pop(NCHUNK - 2):
        d.wait()

    @pl.when(wid < 2)
    def _tail():
        tb = TAIL_BASE + wid * TAIL_CH
        pltpu.sync_copy(gs_hbm.at[pl.ds(tb, TAIL_CH)],
                        gs_b[0].at[pl.ds(0, TAIL_CH)])
        pltpu.sync_copy(gv_hbm.at[pl.ds(tb, TAIL_CH)],
                        gv_b[0].at[pl.ds(0, TAIL_CH)])
        pltpu.sync_copy(idx_hbm.at[pl.ds(tb // 128, 1)],
                        idx_b[0].at[pl.ds(0, 1)])
        compute(TAIL_CH // 16, 0)
        pltpu.sync_copy(rows_b[0].at[pl.ds(0, TAIL_CH)],
                        acc.at[idx_b[0].at[0]], add=True)

    for d in pend_sc.pop(NCHUNK - 1):
        d.wait()
    plsc.subcore_barrier()
    pltpu.sync_copy(acc.at[pl.ds(sid * ROWS_PER_SUB, ROWS_PER_SUB)],
                    out_hbm.at[cid, pl.ds(sid * ROWS_PER_SUB, ROWS_PER_SUB)])


@functools.lru_cache(maxsize=1)
def _sc_edge():
    return pl.kernel(
        _sc_edge_body,
        out_type=jax.ShapeDtypeStruct((2, NPAD, 16), jnp.float32),
        mesh=plsc.VectorSubcoreMesh(core_axis_name="c", subcore_axis_name="s",
                                    num_cores=2, num_subcores=16),
        compiler_params=pltpu.CompilerParams(needs_layout_passes=False,
                                             use_tc_tiling_on_sc=False),
        scratch_types=[
            pltpu.VMEM((CH, 4), jnp.float32),
            pltpu.VMEM((CH, 4), jnp.float32),
            pltpu.VMEM((CH, 12), jnp.float32),
            pltpu.VMEM((CH, 12), jnp.float32),
            pltpu.VMEM((CH // 128, 128), jnp.int32),
            pltpu.VMEM((CH // 128, 128), jnp.int32),
            pltpu.VMEM((CH, 16), jnp.float32),
            pltpu.VMEM((CH, 16), jnp.float32),
            pltpu.VMEM((ROWS_PER_SUB, 16), jnp.float32),
            pltpu.VMEM_SHARED((NPAD, 16), jnp.float32),
            pltpu.SemaphoreType.DMA,
            pltpu.SemaphoreType.DMA,
            pltpu.SemaphoreType.DMA,
            pltpu.SemaphoreType.DMA,
        ],
    )


def _tc_node_body(a_ref, q_ref, acc0_ref, acc1_ref, agh_ref, w1_ref, b1_ref,
                  w2_ref, b2_ref, w3_ref, b3_ref, out_ref):
    a = a_ref[:, :]
    ac = acc0_ref[:, :] + acc1_ref[:, :]
    s = ac[:, 0:1]
    radial = a * s
    rq = q_ref[:, :] * s

    bf = jnp.bfloat16
    t = jnp.dot(a.astype(bf), agh_ref[:, :].astype(bf),
                preferred_element_type=jnp.float32)
    t = t.astype(bf).astype(jnp.float32)
    tg = [t[:, j * 8:(j + 1) * 8] for j in range(4)]
    ve = ac[:, 1:2] * tg[0] * tg[0]
    for j in range(1, 4):
        ve = ve + ac[:, 1 + j:2 + j] * tg[j] * tg[j]
    for col, (j, k) in enumerate(_OFFDIAG):
        ve = ve + 2.0 * ac[:, 5 + col:6 + col] * tg[j] * tg[k]

    w1 = w1_ref[:, :]
    pre1 = (jnp.dot(radial.astype(bf), w1[0:128, :].astype(bf),
                    preferred_element_type=jnp.float32)
            + jnp.dot(ve.astype(bf), w1[128:136, :].astype(bf),
                      preferred_element_type=jnp.float32)
            + rq.astype(bf).astype(jnp.float32) * w1[136:137, :]
            + b1_ref[:, :])
    h1 = jax.nn.gelu(pre1)
    h2 = jax.nn.gelu(jnp.dot(h1.astype(bf), w2_ref[:, :].astype(bf),
                             preferred_element_type=jnp.float32) + b2_ref[:, :])
    out_ref[:, :] = jnp.dot(h2.astype(bf), w3_ref[:, :].astype(bf),
                            preferred_element_type=jnp.float32) + b3_ref[:, :]


def _tc_node(a, q, acc0, acc1, agh2, w1, b1, w2, b2, w3, b3):
    R = 512
    grid = (NPAD // R,)
    return pl.pallas_call(
        _tc_node_body,
        grid=grid,
        in_specs=[
            pl.BlockSpec((R, F), lambda i: (i, 0)),
            pl.BlockSpec((R, 1), lambda i: (i, 0)),
            pl.BlockSpec((R, 16), lambda i: (i, 0)),
            pl.BlockSpec((R, 16), lambda i: (i, 0)),
            pl.BlockSpec((F, G * V), lambda i: (0, 0)),
            pl.BlockSpec((F + 2 * V + 1, F), lambda i: (0, 0)),
            pl.BlockSpec((1, F), lambda i: (0, 0)),
            pl.BlockSpec((F, F), lambda i: (0, 0)),
            pl.BlockSpec((1, F), lambda i: (0, 0)),
            pl.BlockSpec((F, F + 2), lambda i: (0, 0)),
            pl.BlockSpec((1, F + 2), lambda i: (0, 0)),
        ],
        out_specs=pl.BlockSpec((R, F + 2), lambda i: (i, 0)),
        out_shape=jax.ShapeDtypeStruct((NPAD, F + 2), jnp.float32),
    )(a, q, acc0, acc1, agh2, w1, b1, w2, b2, w3, b3)


def kernel(atomic_embedding, partial_charges, pair_indices, gs, gv, agh,
           W1, b1, W2, b2, W3, b3):
    idx_w = pair_indices[1].reshape(E // 128, 128)
    acc = _sc_edge()(idx_w, gs, gv.reshape(E, 12))

    pad_n = NPAD - N
    a_p = jnp.concatenate([atomic_embedding, jnp.zeros((pad_n, F), jnp.float32)])
    q_p = jnp.concatenate([partial_charges, jnp.zeros((pad_n, 1), jnp.float32)])

    out = _tc_node(a_p, q_p, acc[0], acc[1], agh.reshape(F, G * V),
                   W1, b1.reshape(1, F), W2, b2.reshape(1, F),
                   W3, b3.reshape(1, F + 2))
    return (out[:N, 2:F + 2], out[:N, 0:1], out[:N, 1:2])
